# initial kernel scaffold (unmeasured)
import jax
import jax.numpy as jnp
from jax import lax
from jax.experimental import pallas as pl
from jax.experimental.pallas import tpu as pltpu

N_DEV = 16
SQ = 256
SKV = 4096
HQ_LOCAL = 8
DH = 128
DM = 1024
CHUNK = SQ // N_DEV
SCALE = 0.08838834764831843


def kernel(x, Wq, K_ext, V_ext, Wo):
    my = lax.axis_index("i")
    Wq_l = lax.dynamic_slice(Wq, (0, my * DM), (Wq.shape[0], DM))
    Wo_l = lax.dynamic_slice(Wo, (my * DM, 0), (DM, Wo.shape[1]))
    x2 = x.reshape(SQ, DM)
    K = K_ext.reshape(SKV, HQ_LOCAL, DH)
    V = V_ext.reshape(SKV, HQ_LOCAL, DH)

    def body(x_ref, wq_ref, k_ref, v_ref, wo_ref, out_ref,
             q_s, ctx_s, rs_buf, rs_send_sems, rs_recv_sems,
             ag_send_sems, ag_recv_sems):
        my_pos = lax.axis_index("i")
        left = (my_pos - 1) % N_DEV
        right = (my_pos + 1) % N_DEV

        barrier_sem = pltpu.get_barrier_semaphore()
        for nbr in (left, right):
            pl.semaphore_signal(
                barrier_sem, inc=1,
                device_id=(nbr,), device_id_type=pl.DeviceIdType.MESH,
            )
        pl.semaphore_wait(barrier_sem, 2)

        q_s[:, :] = jnp.dot(x_ref[:, :], wq_ref[:, :],
                            preferred_element_type=jnp.float32)

        qi = lax.broadcasted_iota(jnp.int32, (SQ, SKV), 0) // 64
        ki = lax.broadcasted_iota(jnp.int32, (SQ, SKV), 1) // 64
        mask = (qi == ki) | ((ki % 4) == (qi % 4))

        for h in range(HQ_LOCAL):
            qh = q_s[:, h * DH:(h + 1) * DH]
            kh = k_ref[:, h, :]
            s = jnp.dot(qh, kh.T, preferred_element_type=jnp.float32)
            s = jnp.where(mask, s * SCALE, -1e9)
            m = jnp.max(s, axis=1, keepdims=True)
            w = jnp.exp(s - m)
            w = w / jnp.sum(w, axis=1, keepdims=True)
            ctx_s[:, h * DH:(h + 1) * DH] = jnp.dot(
                w, v_ref[:, h, :], preferred_element_type=jnp.float32)

        out_ref[:, :] = jnp.dot(ctx_s[:, :], wo_ref[:, :],
                                preferred_element_type=jnp.float32)

        for s in range(N_DEV - 1):
            send_c = (my_pos - s) % N_DEV
            recv_c = (my_pos - s - 1) % N_DEV
            rdma = pltpu.make_async_remote_copy(
                src_ref=out_ref.at[pl.ds(send_c * CHUNK, CHUNK), :],
                dst_ref=rs_buf.at[s],
                send_sem=rs_send_sems.at[s],
                recv_sem=rs_recv_sems.at[s],
                device_id=(right,),
                device_id_type=pl.DeviceIdType.MESH,
            )
            rdma.start()
            rdma.wait()
            out_ref[pl.ds(recv_c * CHUNK, CHUNK), :] = (
                out_ref[pl.ds(recv_c * CHUNK, CHUNK), :] + rs_buf[s])

        for s in range(N_DEV - 1):
            send_c = (my_pos + 1 - s) % N_DEV
            rdma = pltpu.make_async_remote_copy(
                src_ref=out_ref.at[pl.ds(send_c * CHUNK, CHUNK), :],
                dst_ref=out_ref.at[pl.ds(send_c * CHUNK, CHUNK), :],
                send_sem=ag_send_sems.at[s],
                recv_sem=ag_recv_sems.at[s],
                device_id=(right,),
                device_id_type=pl.DeviceIdType.MESH,
            )
            rdma.start()
            rdma.wait()

    out = pl.pallas_call(
        body,
        out_shape=jax.ShapeDtypeStruct((SQ, Wo.shape[1]), jnp.float32),
        in_specs=[pl.BlockSpec(memory_space=pltpu.VMEM)] * 5,
        out_specs=pl.BlockSpec(memory_space=pltpu.VMEM),
        scratch_shapes=[
            pltpu.VMEM((SQ, DM), jnp.float32),
            pltpu.VMEM((SQ, DM), jnp.float32),
            pltpu.VMEM((N_DEV - 1, CHUNK, DM), jnp.float32),
            pltpu.SemaphoreType.DMA((N_DEV - 1,)),
            pltpu.SemaphoreType.DMA((N_DEV - 1,)),
            pltpu.SemaphoreType.DMA((N_DEV - 1,)),
            pltpu.SemaphoreType.DMA((N_DEV - 1,)),
        ],
        compiler_params=pltpu.CompilerParams(collective_id=0),
    )(x2, Wq_l, K, V, Wo_l)
    return out.reshape(1, SQ, Wo.shape[1])


# baseline (device time: 146146 ns/iter reference)
import jax
import jax.numpy as jnp
from jax import lax
from jax.experimental import pallas as pl
from jax.experimental.pallas import tpu as pltpu

N_DEV = 16
SQ = 256
SKV = 4096
HQ_LOCAL = 8
DH = 128
DM = 1024
QB = 64
NR = 4
KKEPT = SKV // NR
CHUNK = SQ // N_DEV
SCALE = 0.08838834764831843


def _gather_blocks(a):
    a4 = a.reshape(SKV // (NR * QB), NR, QB, HQ_LOCAL, DH)
    return a4.transpose(1, 3, 0, 2, 4).reshape(NR, HQ_LOCAL, KKEPT, DH)


def kernel(x, Wq, K_ext, V_ext, Wo):
    my = lax.axis_index("i")
    Wq_l = lax.dynamic_slice(Wq, (0, my * DM), (Wq.shape[0], DM))
    Wo_l = lax.dynamic_slice(Wo, (my * DM, 0), (DM, Wo.shape[1]))
    x2 = x.reshape(SQ, DM)
    Kg = _gather_blocks(K_ext)
    Vg = _gather_blocks(V_ext)

    def body(x_ref, wq_ref, kg_ref, vg_ref, wo_ref, out_ref,
             q_s, ctx_s, rs_buf, rs_send_sems, rs_recv_sems,
             ag_send_sems, ag_recv_sems):
        my_pos = lax.axis_index("i")
        left = (my_pos - 1) % N_DEV
        right = (my_pos + 1) % N_DEV

        barrier_sem = pltpu.get_barrier_semaphore()
        for nbr in (left, right):
            pl.semaphore_signal(
                barrier_sem, inc=1,
                device_id=(nbr,), device_id_type=pl.DeviceIdType.MESH,
            )
        pl.semaphore_wait(barrier_sem, 2)

        q_s[:, :] = jnp.dot(x_ref[:, :], wq_ref[:, :],
                            preferred_element_type=jnp.float32)

        for h in range(HQ_LOCAL):
            for r in range(NR):
                qrh = q_s[r * QB:(r + 1) * QB, h * DH:(h + 1) * DH]
                s = lax.dot_general(
                    qrh, kg_ref[r, h],
                    (((1,), (1,)), ((), ())),
                    preferred_element_type=jnp.float32,
                ) * SCALE
                m = jnp.max(s, axis=1, keepdims=True)
                w = jnp.exp(s - m)
                w = w / jnp.sum(w, axis=1, keepdims=True)
                ctx_s[r * QB:(r + 1) * QB, h * DH:(h + 1) * DH] = jnp.dot(
                    w, vg_ref[r, h], preferred_element_type=jnp.float32)

        out_ref[:, :] = jnp.dot(ctx_s[:, :], wo_ref[:, :],
                                preferred_element_type=jnp.float32)

        for s in range(N_DEV - 1):
            send_c = (my_pos - s) % N_DEV
            recv_c = (my_pos - s - 1) % N_DEV
            rdma = pltpu.make_async_remote_copy(
                src_ref=out_ref.at[pl.ds(send_c * CHUNK, CHUNK), :],
                dst_ref=rs_buf.at[s],
                send_sem=rs_send_sems.at[s],
                recv_sem=rs_recv_sems.at[s],
                device_id=(right,),
                device_id_type=pl.DeviceIdType.MESH,
            )
            rdma.start()
            rdma.wait()
            out_ref[pl.ds(recv_c * CHUNK, CHUNK), :] = (
                out_ref[pl.ds(recv_c * CHUNK, CHUNK), :] + rs_buf[s])

        for s in range(N_DEV - 1):
            send_c = (my_pos + 1 - s) % N_DEV
            rdma = pltpu.make_async_remote_copy(
                src_ref=out_ref.at[pl.ds(send_c * CHUNK, CHUNK), :],
                dst_ref=out_ref.at[pl.ds(send_c * CHUNK, CHUNK), :],
                send_sem=ag_send_sems.at[s],
                recv_sem=ag_recv_sems.at[s],
                device_id=(right,),
                device_id_type=pl.DeviceIdType.MESH,
            )
            rdma.start()
            rdma.wait()

    out = pl.pallas_call(
        body,
        out_shape=jax.ShapeDtypeStruct((SQ, Wo.shape[1]), jnp.float32),
        in_specs=[pl.BlockSpec(memory_space=pltpu.VMEM)] * 5,
        out_specs=pl.BlockSpec(memory_space=pltpu.VMEM),
        scratch_shapes=[
            pltpu.VMEM((SQ, DM), jnp.float32),
            pltpu.VMEM((SQ, DM), jnp.float32),
            pltpu.VMEM((N_DEV - 1, CHUNK, DM), jnp.float32),
            pltpu.SemaphoreType.DMA((N_DEV - 1,)),
            pltpu.SemaphoreType.DMA((N_DEV - 1,)),
            pltpu.SemaphoreType.DMA((N_DEV - 1,)),
            pltpu.SemaphoreType.DMA((N_DEV - 1,)),
        ],
        compiler_params=pltpu.CompilerParams(
            collective_id=0,
            vmem_limit_bytes=100 * 1024 * 1024,
        ),
    )(x2, Wq_l, Kg, Vg, Wo_l)
    return out.reshape(1, SQ, Wo.shape[1])


# device time: 93493 ns/iter; 1.5632x vs baseline; 1.5632x over previous
import jax
import jax.numpy as jnp
from jax import lax
from jax.experimental import pallas as pl
from jax.experimental.pallas import tpu as pltpu

N_DEV = 16
SQ = 256
SKV = 4096
HQ_LOCAL = 8
DH = 128
DM = 1024
QB = 64
NR = 4
KKEPT = SKV // NR
CHUNK = SQ // N_DEV
SCALE = 0.08838834764831843


def _gather_blocks(a):
    a4 = a.reshape(SKV // (NR * QB), NR, QB, HQ_LOCAL, DH)
    return a4.transpose(1, 3, 0, 2, 4).reshape(NR, HQ_LOCAL, KKEPT, DH)


def kernel(x, Wq, K_ext, V_ext, Wo):
    my = lax.axis_index("i")
    Wq_l = lax.dynamic_slice(Wq, (0, my * DM), (Wq.shape[0], DM))
    Wo_l = lax.dynamic_slice(Wo, (my * DM, 0), (DM, Wo.shape[1]))
    x2 = x.reshape(SQ, DM)
    Kg = _gather_blocks(K_ext)
    Vg = _gather_blocks(V_ext)

    def body(x_ref, wq_ref, kg_ref, vg_ref, wo_ref, out_ref,
             q_s, ctx_s, rs_buf, rs_send_sems, rs_recv_sems,
             ag_send_sems, ag_recv_sems):
        my_pos = lax.axis_index("i")

        barrier_sem = pltpu.get_barrier_semaphore()
        for o in range(1, N_DEV):
            pl.semaphore_signal(
                barrier_sem, inc=1,
                device_id=((my_pos + o) % N_DEV,),
                device_id_type=pl.DeviceIdType.MESH,
            )
        pl.semaphore_wait(barrier_sem, N_DEV - 1)

        q_s[:, :] = jnp.dot(x_ref[:, :], wq_ref[:, :],
                            preferred_element_type=jnp.float32)

        for h in range(HQ_LOCAL):
            for r in range(NR):
                qrh = q_s[r * QB:(r + 1) * QB, h * DH:(h + 1) * DH]
                s = lax.dot_general(
                    qrh, kg_ref[r, h],
                    (((1,), (1,)), ((), ())),
                    preferred_element_type=jnp.float32,
                ) * SCALE
                m = jnp.max(s, axis=1, keepdims=True)
                w = jnp.exp(s - m)
                w = w / jnp.sum(w, axis=1, keepdims=True)
                ctx_s[r * QB:(r + 1) * QB, h * DH:(h + 1) * DH] = jnp.dot(
                    w, vg_ref[r, h], preferred_element_type=jnp.float32)

        out_ref[:, :] = jnp.dot(ctx_s[:, :], wo_ref[:, :],
                                preferred_element_type=jnp.float32)

        for o in range(1, N_DEV):
            tgt = (my_pos + o) % N_DEV
            rdma = pltpu.make_async_remote_copy(
                src_ref=out_ref.at[pl.ds(tgt * CHUNK, CHUNK), :],
                dst_ref=rs_buf.at[o - 1],
                send_sem=rs_send_sems.at[o - 1],
                recv_sem=rs_recv_sems.at[o - 1],
                device_id=(tgt,),
                device_id_type=pl.DeviceIdType.MESH,
            )
            rdma.start()
        for j in range(N_DEV - 1):
            recv = pltpu.make_async_remote_copy(
                src_ref=rs_buf.at[j],
                dst_ref=rs_buf.at[j],
                send_sem=rs_send_sems.at[j],
                recv_sem=rs_recv_sems.at[j],
                device_id=(my_pos,),
                device_id_type=pl.DeviceIdType.MESH,
            )
            recv.wait_recv()
        acc = out_ref[pl.ds(my_pos * CHUNK, CHUNK), :]
        acc = acc + jnp.sum(rs_buf[:, :, :], axis=0)
        out_ref[pl.ds(my_pos * CHUNK, CHUNK), :] = acc

        for o in range(1, N_DEV):
            tgt = (my_pos + o) % N_DEV
            rdma = pltpu.make_async_remote_copy(
                src_ref=out_ref.at[pl.ds(my_pos * CHUNK, CHUNK), :],
                dst_ref=out_ref.at[pl.ds(my_pos * CHUNK, CHUNK), :],
                send_sem=ag_send_sems.at[o - 1],
                recv_sem=ag_recv_sems.at[o - 1],
                device_id=(tgt,),
                device_id_type=pl.DeviceIdType.MESH,
            )
            rdma.start()
        for j in range(N_DEV - 1):
            src_dev = (my_pos - j - 1) % N_DEV
            recv = pltpu.make_async_remote_copy(
                src_ref=out_ref.at[pl.ds(src_dev * CHUNK, CHUNK), :],
                dst_ref=out_ref.at[pl.ds(src_dev * CHUNK, CHUNK), :],
                send_sem=ag_send_sems.at[j],
                recv_sem=ag_recv_sems.at[j],
                device_id=(my_pos,),
                device_id_type=pl.DeviceIdType.MESH,
            )
            recv.wait_recv()
        for j in range(N_DEV - 1):
            for sems in (rs_send_sems, ag_send_sems):
                drain = pltpu.make_async_remote_copy(
                    src_ref=rs_buf.at[j],
                    dst_ref=rs_buf.at[j],
                    send_sem=sems.at[j],
                    recv_sem=rs_recv_sems.at[j],
                    device_id=(my_pos,),
                    device_id_type=pl.DeviceIdType.MESH,
                )
                drain.wait_send()

    out = pl.pallas_call(
        body,
        out_shape=jax.ShapeDtypeStruct((SQ, Wo.shape[1]), jnp.float32),
        in_specs=[pl.BlockSpec(memory_space=pltpu.VMEM)] * 5,
        out_specs=pl.BlockSpec(memory_space=pltpu.VMEM),
        scratch_shapes=[
            pltpu.VMEM((SQ, DM), jnp.float32),
            pltpu.VMEM((SQ, DM), jnp.float32),
            pltpu.VMEM((N_DEV - 1, CHUNK, DM), jnp.float32),
            pltpu.SemaphoreType.DMA((N_DEV - 1,)),
            pltpu.SemaphoreType.DMA((N_DEV - 1,)),
            pltpu.SemaphoreType.DMA((N_DEV - 1,)),
            pltpu.SemaphoreType.DMA((N_DEV - 1,)),
        ],
        compiler_params=pltpu.CompilerParams(
            collective_id=0,
            vmem_limit_bytes=100 * 1024 * 1024,
        ),
    )(x2, Wq_l, Kg, Vg, Wo_l)
    return out.reshape(1, SQ, Wo.shape[1])
